# baseline (speedup 1.0000x reference)
import functools

import jax
import jax.numpy as jnp
from jax import lax
from jax.experimental import pallas as pl
from jax.experimental.pallas import tpu as pltpu
from jax.experimental.pallas import tpu_sc as plsc

NC = 2
NS = 16
NW = NC * NS
L = 16
EPS = 1e-5


def _rsqrt(x):
    i = lax.bitcast_convert_type(x, jnp.int32)
    i = jnp.int32(0x5F3759DF) - lax.shift_right_logical(i, 1)
    y = lax.bitcast_convert_type(i, jnp.float32)
    for _ in range(3):
        y = y * (1.5 - 0.5 * x * y * y)
    return y


def kernel(input_ids, token_table, pos_table, gamma, beta):
    B, S = input_ids.shape
    V, D = token_table.shape
    assert B % NW == 0 and D == 2 * L
    spw = B // NW

    @functools.partial(
        pl.kernel,
        out_type=jax.ShapeDtypeStruct((B, S, D), jnp.float32),
        mesh=plsc.VectorSubcoreMesh(core_axis_name="c", subcore_axis_name="s"),
        compiler_params=pltpu.CompilerParams(
            needs_layout_passes=False, use_tc_tiling_on_sc=False),
        scratch_types=[
            pltpu.VMEM((spw, S), jnp.int32),
            pltpu.VMEM((S, D), jnp.float32),
            pltpu.VMEM((S, D), jnp.float32),
            pltpu.VMEM((S, D), jnp.float32),
            pltpu.VMEM((D,), jnp.float32),
            pltpu.VMEM((D,), jnp.float32),
            pltpu.SemaphoreType.DMA,
        ],
    )
    def run(ids_hbm, tok_hbm, pos_hbm, g_hbm, b_hbm, out_hbm,
            idx_v, rows_v, pos_v, outb_v, g_v, b_v, sem):
        wid = lax.axis_index("s") * NC + lax.axis_index("c")
        base = wid * spw
        pltpu.sync_copy(ids_hbm.at[pl.ds(base, spw), :], idx_v)
        pltpu.sync_copy(pos_hbm, pos_v)
        pltpu.sync_copy(g_hbm, g_v)
        pltpu.sync_copy(b_hbm, b_v)
        g0 = g_v[0:L]
        g1 = g_v[L:D]
        bt0 = b_v[0:L]
        bt1 = b_v[L:D]

        def seq_body(i, carry):
            pltpu.async_copy(tok_hbm.at[idx_v.at[i]], rows_v, sem).wait()

            def tok_body(j, c2):
                x0 = rows_v[j, 0:L] + pos_v[j, 0:L]
                x1 = rows_v[j, L:D] + pos_v[j, L:D]
                m = jnp.sum(x0 + x1) * (1.0 / D)
                d0 = x0 - m
                d1 = x1 - m
                var = jnp.sum(d0 * d0 + d1 * d1) * (1.0 / D)
                r = _rsqrt(var + EPS)
                outb_v[j, 0:L] = d0 * r * g0 + bt0
                outb_v[j, L:D] = d1 * r * g1 + bt1
                return c2

            lax.fori_loop(0, S, tok_body, 0, unroll=8)
            pltpu.sync_copy(outb_v, out_hbm.at[base + i])
            return carry

        lax.fori_loop(0, spw, seq_body, 0)

    return run(input_ids.astype(jnp.int32), token_table, pos_table, gamma, beta)
